# Initial kernel scaffold; baseline (speedup 1.0000x reference)
import jax
import jax.numpy as jnp
from jax.experimental import pallas as pl


def kernel(features, edge_index, positive_edge_pairs, negative_edge_pairs, W1, b1, W2, b2):
    raise NotImplementedError("write your pallas kernel here")


# speedup vs baseline: 6.7905x; 6.7905x over previous
import functools

import jax
import jax.numpy as jnp
from jax import lax
from jax.experimental import pallas as pl
from jax.experimental.pallas import tpu as pltpu
from jax.experimental.pallas import tpu_sc as plsc

N_TILES = 16
N_CORES = 2
NW = N_TILES * N_CORES
CHUNK = 128


def _tc_matmul_bias(x, w, b, block_rows=512):
    M, K = x.shape
    D = w.shape[1]

    def body(x_ref, w_ref, b_ref, o_ref):
        o_ref[...] = (
            jnp.dot(x_ref[...], w_ref[...], preferred_element_type=jnp.float32)
            + b_ref[...]
        )

    return pl.pallas_call(
        body,
        out_shape=jax.ShapeDtypeStruct((M, D), jnp.float32),
        grid=(M // block_rows,),
        in_specs=[
            pl.BlockSpec((block_rows, K), lambda i: (i, 0)),
            pl.BlockSpec((K, D), lambda i: (0, 0)),
            pl.BlockSpec((1, D), lambda i: (0, 0)),
        ],
        out_specs=pl.BlockSpec((block_rows, D), lambda i: (i, 0)),
    )(x, w, b)


def _tc_relu_add_matmul_bias(p0, p1, w, b, block_rows=512):
    M, K = p0.shape
    D = w.shape[1]

    def body(p0_ref, p1_ref, w_ref, b_ref, o_ref):
        h = jnp.maximum(p0_ref[...] + p1_ref[...], 0.0)
        o_ref[...] = (
            jnp.dot(h, w_ref[...], preferred_element_type=jnp.float32) + b_ref[...]
        )

    return pl.pallas_call(
        body,
        out_shape=jax.ShapeDtypeStruct((M, D), jnp.float32),
        grid=(M // block_rows,),
        in_specs=[
            pl.BlockSpec((block_rows, K), lambda i: (i, 0)),
            pl.BlockSpec((block_rows, K), lambda i: (i, 0)),
            pl.BlockSpec((K, D), lambda i: (0, 0)),
            pl.BlockSpec((1, D), lambda i: (0, 0)),
        ],
        out_specs=pl.BlockSpec((block_rows, D), lambda i: (i, 0)),
    )(p0, p1, w, b)


def _tc_add(p0, p1, block_rows=512):
    M, D = p0.shape

    def body(a_ref, b_ref, o_ref):
        o_ref[...] = a_ref[...] + b_ref[...]

    return pl.pallas_call(
        body,
        out_shape=jax.ShapeDtypeStruct((M, D), jnp.float32),
        grid=(M // block_rows,),
        in_specs=[
            pl.BlockSpec((block_rows, D), lambda i: (i, 0)),
            pl.BlockSpec((block_rows, D), lambda i: (i, 0)),
        ],
        out_specs=pl.BlockSpec((block_rows, D), lambda i: (i, 0)),
    )(p0, p1)


def _sc_segment_sum(hw, edges3d, zeros_np, n_chunks_per_tile):
    NP, D = hw.shape
    rows_per_tile = NP // N_TILES
    mesh = plsc.VectorSubcoreMesh(core_axis_name="c", subcore_axis_name="s")

    @functools.partial(
        pl.kernel,
        out_type=jax.ShapeDtypeStruct((N_CORES, NP, D), jnp.float32),
        mesh=mesh,
        scratch_types=[
            pltpu.VMEM((2, CHUNK), jnp.int32),
            pltpu.VMEM((CHUNK, D), jnp.float32),
            pltpu.VMEM_SHARED((NP, D), jnp.float32),
        ],
        compiler_params=pltpu.CompilerParams(use_tc_tiling_on_sc=False),
    )
    def k(hw_hbm, edges_hbm, zeros_hbm, out_hbm, idx_v, rows_v, acc):
        cid = lax.axis_index("c")
        sid = lax.axis_index("s")
        wid = sid * N_CORES + cid
        r0 = sid * rows_per_tile
        pltpu.sync_copy(
            zeros_hbm.at[pl.ds(r0, rows_per_tile)],
            acc.at[pl.ds(r0, rows_per_tile)],
        )
        plsc.subcore_barrier()

        base = wid * n_chunks_per_tile

        def body(j, carry):
            row = base + j
            pltpu.sync_copy(edges_hbm.at[row], idx_v)
            pltpu.sync_copy(hw_hbm.at[idx_v.at[0]], rows_v)
            pltpu.sync_copy(rows_v, acc.at[idx_v.at[1]], add=True)
            return carry

        lax.fori_loop(0, n_chunks_per_tile, body, 0)
        plsc.subcore_barrier()
        pltpu.sync_copy(
            acc.at[pl.ds(r0, rows_per_tile)],
            out_hbm.at[cid, pl.ds(r0, rows_per_tile)],
        )

    return k(hw, edges3d, zeros_np)


def _sc_decoder(h2, pairs3d, n_chunks_per_tile):
    NP, D = h2.shape
    CH = pairs3d.shape[0]
    mesh = plsc.VectorSubcoreMesh(core_axis_name="c", subcore_axis_name="s")

    @functools.partial(
        pl.kernel,
        out_type=jax.ShapeDtypeStruct((CH * CHUNK,), jnp.float32),
        mesh=mesh,
        scratch_types=[
            pltpu.VMEM((2, CHUNK), jnp.int32),
            pltpu.VMEM((CHUNK, D), jnp.float32),
            pltpu.VMEM((CHUNK, D), jnp.float32),
            pltpu.VMEM((CHUNK,), jnp.float32),
        ],
        compiler_params=pltpu.CompilerParams(
            use_tc_tiling_on_sc=False, needs_layout_passes=False
        ),
    )
    def k(h_hbm, pairs_hbm, out_hbm, idx_v, ra, rb, res_v):
        cid = lax.axis_index("c")
        sid = lax.axis_index("s")
        wid = sid * N_CORES + cid
        base = wid * n_chunks_per_tile

        lane = lax.iota(jnp.int32, 16)

        def body(j, carry):
            row = base + j
            pltpu.sync_copy(pairs_hbm.at[row], idx_v)
            pltpu.sync_copy(h_hbm.at[idx_v.at[0]], ra)
            pltpu.sync_copy(h_hbm.at[idx_v.at[1]], rb)

            def group_body(g, c2):
                res = jnp.zeros((16,), jnp.float32)
                for p2 in range(16):
                    p = g * 16 + p2
                    acc = ra[p, pl.ds(0, 16)] * rb[p, pl.ds(0, 16)]
                    acc = acc + ra[p, pl.ds(16, 16)] * rb[p, pl.ds(16, 16)]
                    acc = acc + ra[p, pl.ds(32, 16)] * rb[p, pl.ds(32, 16)]
                    acc = acc + ra[p, pl.ds(48, 16)] * rb[p, pl.ds(48, 16)]
                    res = jnp.where(lane == p2, jnp.sum(acc), res)
                res_v[pl.ds(g * 16, 16)] = res
                return c2

            lax.fori_loop(0, CHUNK // 16, group_body, 0)
            pltpu.sync_copy(res_v, out_hbm.at[pl.ds(row * CHUNK, CHUNK)])
            return carry

        lax.fori_loop(0, n_chunks_per_tile, body, 0)

    return k(h2, pairs3d)


def _pad_edges(src, dst, n_real_rows, n_pad_rows):
    E = src.shape[0]
    round_sz = NW * CHUNK
    Epad = ((E + round_sz - 1) // round_sz) * round_sz
    pad = Epad - E
    if pad:
        ar = jnp.arange(pad, dtype=jnp.int32)
        src = jnp.concatenate([src, (ar * 97) % n_real_rows])
        dst = jnp.concatenate([dst, n_real_rows + ar % n_pad_rows])
    edges3d = jnp.stack(
        [src.reshape(-1, CHUNK), dst.reshape(-1, CHUNK)], axis=1
    )
    return edges3d, Epad


def kernel(features, edge_index, positive_edge_pairs, negative_edge_pairs,
           W1, b1, W2, b2):
    N, D_IN = features.shape
    H1 = W1.shape[1]
    H2 = W2.shape[1]

    NP = 10752
    n_pad_rows = NP - N

    xp = jnp.concatenate(
        [features, jnp.zeros((NP - N, D_IN), jnp.float32)], axis=0
    )
    src = edge_index[0].astype(jnp.int32)
    dst = edge_index[1].astype(jnp.int32)
    edges3d, Epad = _pad_edges(src, dst, N, n_pad_rows)
    n_chunks_per_tile = Epad // (NW * CHUNK)

    z1 = jnp.zeros((NP, H1), jnp.float32)
    z2 = jnp.zeros((NP, H2), jnp.float32)

    hw1 = _tc_matmul_bias(xp, W1, b1.reshape(1, H1))
    parts1 = _sc_segment_sum(hw1, edges3d, z1, n_chunks_per_tile)

    hw2 = _tc_relu_add_matmul_bias(parts1[0], parts1[1], W2, b2.reshape(1, H2))
    parts2 = _sc_segment_sum(hw2, edges3d, z2, n_chunks_per_tile)
    h2 = _tc_add(parts2[0], parts2[1])

    pa = jnp.concatenate(
        [positive_edge_pairs[0], negative_edge_pairs[0]]
    ).astype(jnp.int32)
    pb = jnp.concatenate(
        [positive_edge_pairs[1], negative_edge_pairs[1]]
    ).astype(jnp.int32)
    P = pa.shape[0]
    round_sz = NW * CHUNK
    Ppad = ((P + round_sz - 1) // round_sz) * round_sz
    padp = Ppad - P
    if padp:
        ar = jnp.arange(padp, dtype=jnp.int32)
        pa = jnp.concatenate([pa, (ar * 131) % N])
        pb = jnp.concatenate([pb, (ar * 173) % N])
    pairs3d = jnp.stack([pa.reshape(-1, CHUNK), pb.reshape(-1, CHUNK)], axis=1)

    result = _sc_decoder(h2, pairs3d, Ppad // (NW * CHUNK))
    return result[:P]


# speedup vs baseline: 10.8017x; 1.5907x over previous
import functools

import jax
import jax.numpy as jnp
from jax import lax
from jax.experimental import pallas as pl
from jax.experimental.pallas import tpu as pltpu
from jax.experimental.pallas import tpu_sc as plsc

N_TILES = 16
N_CORES = 2
NW = N_TILES * N_CORES
CHUNK = 128


def _tc_matmul_bias(x, w, b, block_rows=512):
    M, K = x.shape
    D = w.shape[1]

    def body(x_ref, w_ref, b_ref, o_ref):
        o_ref[...] = (
            jnp.dot(x_ref[...], w_ref[...], preferred_element_type=jnp.float32)
            + b_ref[...]
        )

    return pl.pallas_call(
        body,
        out_shape=jax.ShapeDtypeStruct((M, D), jnp.float32),
        grid=(M // block_rows,),
        in_specs=[
            pl.BlockSpec((block_rows, K), lambda i: (i, 0)),
            pl.BlockSpec((K, D), lambda i: (0, 0)),
            pl.BlockSpec((1, D), lambda i: (0, 0)),
        ],
        out_specs=pl.BlockSpec((block_rows, D), lambda i: (i, 0)),
    )(x, w, b)


def _tc_relu_add_matmul_bias(p0, p1, w, b, block_rows=512):
    M, K = p0.shape
    D = w.shape[1]

    def body(p0_ref, p1_ref, w_ref, b_ref, o_ref):
        h = jnp.maximum(p0_ref[...] + p1_ref[...], 0.0)
        o_ref[...] = (
            jnp.dot(h, w_ref[...], preferred_element_type=jnp.float32) + b_ref[...]
        )

    return pl.pallas_call(
        body,
        out_shape=jax.ShapeDtypeStruct((M, D), jnp.float32),
        grid=(M // block_rows,),
        in_specs=[
            pl.BlockSpec((block_rows, K), lambda i: (i, 0)),
            pl.BlockSpec((block_rows, K), lambda i: (i, 0)),
            pl.BlockSpec((K, D), lambda i: (0, 0)),
            pl.BlockSpec((1, D), lambda i: (0, 0)),
        ],
        out_specs=pl.BlockSpec((block_rows, D), lambda i: (i, 0)),
    )(p0, p1, w, b)


def _tc_add(p0, p1, block_rows=512):
    M, D = p0.shape

    def body(a_ref, b_ref, o_ref):
        o_ref[...] = a_ref[...] + b_ref[...]

    return pl.pallas_call(
        body,
        out_shape=jax.ShapeDtypeStruct((M, D), jnp.float32),
        grid=(M // block_rows,),
        in_specs=[
            pl.BlockSpec((block_rows, D), lambda i: (i, 0)),
            pl.BlockSpec((block_rows, D), lambda i: (i, 0)),
        ],
        out_specs=pl.BlockSpec((block_rows, D), lambda i: (i, 0)),
    )(p0, p1)


def _sc_segment_sum(hw, edges3d, zeros_np, n_chunks_per_tile):
    NP, D = hw.shape
    NCH = n_chunks_per_tile
    GRP = 16
    assert NCH % GRP == 0
    rows_per_tile = NP // N_TILES
    mesh = plsc.VectorSubcoreMesh(core_axis_name="c", subcore_axis_name="s")

    @functools.partial(
        pl.kernel,
        out_type=jax.ShapeDtypeStruct((N_CORES, NP, D), jnp.float32),
        mesh=mesh,
        scratch_types=[
            pltpu.VMEM((GRP, 2, CHUNK), jnp.int32),
            pltpu.VMEM((CHUNK, D), jnp.float32),
            pltpu.VMEM((CHUNK, D), jnp.float32),
            pltpu.VMEM_SHARED((NP, D), jnp.float32),
            pltpu.SemaphoreType.DMA,
            pltpu.SemaphoreType.DMA,
        ],
        compiler_params=pltpu.CompilerParams(use_tc_tiling_on_sc=False),
    )
    def k(hw_hbm, edges_hbm, zeros_hbm, out_hbm, idx_g, rows0, rows1,
          acc, sem0, sem1):
        cid = lax.axis_index("c")
        sid = lax.axis_index("s")
        wid = sid * N_CORES + cid
        r0 = sid * rows_per_tile
        pltpu.sync_copy(
            zeros_hbm.at[pl.ds(r0, rows_per_tile)],
            acc.at[pl.ds(r0, rows_per_tile)],
        )
        plsc.subcore_barrier()

        def group_body(g, carry):
            pltpu.sync_copy(
                edges_hbm.at[pl.ds(wid * NCH + g * GRP, GRP)], idx_g
            )
            pltpu.async_copy(hw_hbm.at[idx_g.at[0, 0]], rows0, sem0)

            def pair_body(k2, c2):
                j = 2 * k2
                pltpu.async_copy(hw_hbm.at[idx_g.at[j + 1, 0]], rows1, sem1)
                pltpu.make_async_copy(
                    hw_hbm.at[idx_g.at[j, 0]], rows0, sem0
                ).wait()
                pltpu.sync_copy(rows0, acc.at[idx_g.at[j, 1]], add=True)

                @pl.when(k2 + 1 < GRP // 2)
                def _():
                    pltpu.async_copy(
                        hw_hbm.at[idx_g.at[j + 2, 0]], rows0, sem0
                    )

                pltpu.make_async_copy(
                    hw_hbm.at[idx_g.at[j + 1, 0]], rows1, sem1
                ).wait()
                pltpu.sync_copy(rows1, acc.at[idx_g.at[j + 1, 1]], add=True)
                return c2

            lax.fori_loop(0, GRP // 2, pair_body, 0)
            return carry

        lax.fori_loop(0, NCH // GRP, group_body, 0)
        plsc.subcore_barrier()
        pltpu.sync_copy(
            acc.at[pl.ds(r0, rows_per_tile)],
            out_hbm.at[cid, pl.ds(r0, rows_per_tile)],
        )

    return k(hw, edges3d, zeros_np)


def _sc_decoder(h2, pairs3d, n_chunks_per_tile):
    NP, D = h2.shape
    CH = pairs3d.shape[0]
    mesh = plsc.VectorSubcoreMesh(core_axis_name="c", subcore_axis_name="s")

    NCH = n_chunks_per_tile
    assert NCH % 2 == 0

    @functools.partial(
        pl.kernel,
        out_type=jax.ShapeDtypeStruct((CH * CHUNK,), jnp.float32),
        mesh=mesh,
        scratch_types=[
            pltpu.VMEM((NCH, 2, CHUNK), jnp.int32),
            pltpu.VMEM((2, CHUNK, D), jnp.float32),
            pltpu.VMEM((2, CHUNK, D), jnp.float32),
            pltpu.VMEM((CHUNK,), jnp.float32),
            pltpu.SemaphoreType.DMA,
            pltpu.SemaphoreType.DMA,
        ],
        compiler_params=pltpu.CompilerParams(
            use_tc_tiling_on_sc=False, needs_layout_passes=False
        ),
    )
    def k(h_hbm, pairs_hbm, out_hbm, idx_all, ra, rb, res_v, sem0, sem1):
        cid = lax.axis_index("c")
        sid = lax.axis_index("s")
        wid = sid * N_CORES + cid
        base = wid * NCH

        lane = lax.iota(jnp.int32, 16)
        pltpu.sync_copy(pairs_hbm.at[pl.ds(base, NCH)], idx_all)

        def start_gather(j, buf, sem):
            pltpu.async_copy(h_hbm.at[idx_all.at[j, 0]], ra.at[buf], sem)
            pltpu.async_copy(h_hbm.at[idx_all.at[j, 1]], rb.at[buf], sem)

        def drain_gather(j, buf, sem):
            pltpu.make_async_copy(
                h_hbm.at[idx_all.at[j, 0]], ra.at[buf], sem
            ).wait()
            pltpu.make_async_copy(
                h_hbm.at[idx_all.at[j, 1]], rb.at[buf], sem
            ).wait()

        def compute(j, buf):
            def group_body(g, c2):
                res = jnp.zeros((16,), jnp.float32)
                for p2 in range(16):
                    p = g * 16 + p2
                    acc = ra[buf, p, pl.ds(0, 16)] * rb[buf, p, pl.ds(0, 16)]
                    acc = acc + ra[buf, p, pl.ds(16, 16)] * rb[buf, p, pl.ds(16, 16)]
                    acc = acc + ra[buf, p, pl.ds(32, 16)] * rb[buf, p, pl.ds(32, 16)]
                    acc = acc + ra[buf, p, pl.ds(48, 16)] * rb[buf, p, pl.ds(48, 16)]
                    res = jnp.where(lane == p2, jnp.sum(acc), res)
                res_v[pl.ds(g * 16, 16)] = res
                return c2

            lax.fori_loop(0, CHUNK // 16, group_body, 0)
            pltpu.sync_copy(
                res_v, out_hbm.at[pl.ds((base + j) * CHUNK, CHUNK)]
            )

        start_gather(0, 0, sem0)

        def body(k2, carry):
            j = 2 * k2
            start_gather(j + 1, 1, sem1)
            drain_gather(j, 0, sem0)
            compute(j, 0)

            @pl.when(k2 + 1 < NCH // 2)
            def _():
                start_gather(j + 2, 0, sem0)

            drain_gather(j + 1, 1, sem1)
            compute(j + 1, 1)
            return carry

        lax.fori_loop(0, NCH // 2, body, 0)

    return k(h2, pairs3d)


def _pad_edges(src, dst, n_real_rows, n_pad_rows):
    E = src.shape[0]
    round_sz = 2 * NW * CHUNK
    Epad = ((E + round_sz - 1) // round_sz) * round_sz
    pad = Epad - E
    if pad:
        ar = jnp.arange(pad, dtype=jnp.int32)
        src = jnp.concatenate([src, (ar * 97) % n_real_rows])
        dst = jnp.concatenate([dst, n_real_rows + ar % n_pad_rows])
    edges3d = jnp.stack(
        [src.reshape(-1, CHUNK), dst.reshape(-1, CHUNK)], axis=1
    )
    return edges3d, Epad


def kernel(features, edge_index, positive_edge_pairs, negative_edge_pairs,
           W1, b1, W2, b2):
    N, D_IN = features.shape
    H1 = W1.shape[1]
    H2 = W2.shape[1]

    NP = 10752
    n_pad_rows = NP - N

    xp = jnp.concatenate(
        [features, jnp.zeros((NP - N, D_IN), jnp.float32)], axis=0
    )
    src = edge_index[0].astype(jnp.int32)
    dst = edge_index[1].astype(jnp.int32)
    edges3d, Epad = _pad_edges(src, dst, N, n_pad_rows)
    n_chunks_per_tile = Epad // (NW * CHUNK)

    z1 = jnp.zeros((NP, H1), jnp.float32)
    z2 = jnp.zeros((NP, H2), jnp.float32)

    hw1 = _tc_matmul_bias(xp, W1, b1.reshape(1, H1))
    parts1 = _sc_segment_sum(hw1, edges3d, z1, n_chunks_per_tile)

    hw2 = _tc_relu_add_matmul_bias(parts1[0], parts1[1], W2, b2.reshape(1, H2))
    parts2 = _sc_segment_sum(hw2, edges3d, z2, n_chunks_per_tile)
    h2 = _tc_add(parts2[0], parts2[1])

    pa = jnp.concatenate(
        [positive_edge_pairs[0], negative_edge_pairs[0]]
    ).astype(jnp.int32)
    pb = jnp.concatenate(
        [positive_edge_pairs[1], negative_edge_pairs[1]]
    ).astype(jnp.int32)
    P = pa.shape[0]
    round_sz = NW * CHUNK
    Ppad = ((P + round_sz - 1) // round_sz) * round_sz
    padp = Ppad - P
    if padp:
        ar = jnp.arange(padp, dtype=jnp.int32)
        pa = jnp.concatenate([pa, (ar * 131) % N])
        pb = jnp.concatenate([pb, (ar * 173) % N])
    pairs3d = jnp.stack([pa.reshape(-1, CHUNK), pb.reshape(-1, CHUNK)], axis=1)

    result = _sc_decoder(h2, pairs3d, Ppad // (NW * CHUNK))
    return result[:P]
